# baseline (device time: 29171 ns/iter reference)
import jax
import jax.numpy as jnp
from jax import lax
from jax.experimental import pallas as pl
from jax.experimental.pallas import tpu as pltpu

N_DEV = 8
BLK = 128

_NEAR_FIRST = (1, 3, 4, 2, 5, 6, 7)


def kernel(x, router_W, route_idx, expert_W, shared_W):
    n_tok, d = x.shape
    e_loc, _, h = expert_W.shape

    def body(x_ref, rw_ref, idx_ref, ew_ref, sw_ref, out_ref,
             sbuf_ref, stage_ref, gath_ref,
             rs_send, rs_recv, ag_send, ag_recv):
        my = lax.axis_index("i")

        barrier = pltpu.get_barrier_semaphore()
        for k in range(1, N_DEV):
            pl.semaphore_signal(
                barrier, inc=1, device_id=(jnp.bitwise_xor(my, k),),
                device_id_type=pl.DeviceIdType.MESH)
        pl.semaphore_wait(barrier, N_DEV - 1)

        rwb = rw_ref[:, :].astype(jnp.bfloat16)
        ewb = [ew_ref[j].astype(jnp.bfloat16) for j in range(e_loc)]

        def block_partial(off):
            xf = x_ref[pl.ds(off, BLK), :]
            xb = xf.astype(jnp.bfloat16)
            sc = jnp.dot(xb, rwb, preferred_element_type=jnp.float32)
            m = jnp.max(sc, axis=1, keepdims=True)
            p = jnp.exp(sc - m)
            p = p / jnp.sum(p, axis=1, keepdims=True)
            col = lax.broadcasted_iota(jnp.int32, p.shape, 1)
            idx = idx_ref[pl.ds(off, BLK), :]
            acc = jnp.zeros((BLK, h), jnp.float32)
            for j in range(e_loc):
                gid = my * e_loc + j
                pj = jnp.sum(jnp.where(col == gid, p, 0.0),
                             axis=1, keepdims=True)
                coef = jnp.where(idx == gid, pj, 0.0)
                acc = acc + jnp.dot((xf * coef).astype(jnp.bfloat16),
                                    ewb[j],
                                    preferred_element_type=jnp.float32)
            return acc

        rs = {}
        for k in range(N_DEV - 1, 0, -1):
            peer = jnp.bitwise_xor(my, k)
            sbuf_ref[k - 1, :, :] = block_partial(peer * BLK).astype(
                jnp.bfloat16)
            rs[k] = pltpu.make_async_remote_copy(
                src_ref=sbuf_ref.at[k - 1],
                dst_ref=stage_ref.at[k - 1],
                send_sem=rs_send.at[k - 1], recv_sem=rs_recv.at[k - 1],
                device_id=(peer,), device_id_type=pl.DeviceIdType.MESH)
            rs[k].start()

        own = block_partial(my * BLK)

        shared = jnp.dot(x_ref[:, :].astype(jnp.bfloat16),
                         sw_ref[:, :].astype(jnp.bfloat16),
                         preferred_element_type=jnp.float32)
        out_ref[:, :] = shared

        red = own.astype(jnp.bfloat16)
        for k in _NEAR_FIRST:
            rs[k].wait()
            red = red + stage_ref[k - 1, :, :]

        gath_ref[pl.ds(my * BLK, BLK), :] = red
        ag = {}
        for k in range(N_DEV - 1, 0, -1):
            ag[k] = pltpu.make_async_remote_copy(
                src_ref=gath_ref.at[pl.ds(my * BLK, BLK)],
                dst_ref=gath_ref.at[pl.ds(my * BLK, BLK)],
                send_sem=ag_send.at[k - 1], recv_sem=ag_recv.at[k - 1],
                device_id=(jnp.bitwise_xor(my, k),),
                device_id_type=pl.DeviceIdType.MESH)
            ag[k].start()

        out_ref[pl.ds(my * BLK, BLK), :] = (
            out_ref[pl.ds(my * BLK, BLK), :] + red.astype(jnp.float32))

        for k in _NEAR_FIRST:
            ag[k].wait()
            boff = jnp.bitwise_xor(my, k) * BLK
            out_ref[pl.ds(boff, BLK), :] = (
                out_ref[pl.ds(boff, BLK), :]
                + gath_ref[pl.ds(boff, BLK), :].astype(jnp.float32))

    return pl.pallas_call(
        body,
        out_shape=jax.ShapeDtypeStruct((n_tok, h), jnp.float32),
        in_specs=[pl.BlockSpec(memory_space=pltpu.VMEM)] * 5,
        out_specs=pl.BlockSpec(memory_space=pltpu.VMEM),
        scratch_shapes=[
            pltpu.VMEM((N_DEV - 1, BLK, h), jnp.bfloat16),
            pltpu.VMEM((N_DEV - 1, BLK, h), jnp.bfloat16),
            pltpu.VMEM((n_tok, h), jnp.bfloat16),
            pltpu.SemaphoreType.DMA((N_DEV - 1,)),
            pltpu.SemaphoreType.DMA((N_DEV - 1,)),
            pltpu.SemaphoreType.DMA((N_DEV - 1,)),
            pltpu.SemaphoreType.DMA((N_DEV - 1,)),
        ],
        compiler_params=pltpu.CompilerParams(collective_id=0),
    )(x, router_W, route_idx, expert_W, shared_W)
